# baseline (speedup 1.0000x reference)
import functools

import jax
import jax.numpy as jnp
from jax import lax
from jax.experimental import pallas as pl
from jax.experimental.pallas import tpu as pltpu


def _gemm_kernel(x_ref, w_ref, o_ref, wb_ref):
    K = wb_ref.shape[0]
    h = K // 2

    @pl.when(pl.program_id(1) == 0)
    def _cast_weight():
        wb_ref[...] = w_ref[...].astype(jnp.bfloat16)

    x_bf = x_ref[...].astype(jnp.bfloat16)
    o_ref[:, :h] = lax.dot_general(
        x_bf,
        wb_ref[:h, :],
        dimension_numbers=(((1,), (1,)), ((), ())),
        preferred_element_type=jnp.float32,
    )
    o_ref[:, h:] = lax.dot_general(
        x_bf,
        wb_ref[h:, :],
        dimension_numbers=(((1,), (1,)), ((), ())),
        preferred_element_type=jnp.float32,
    )


@functools.partial(jax.jit, static_argnames=("tm",))
def _solve(xp, m_param, tm=512):
    B, M, N = xp.shape
    K = m_param.shape[0]
    rows = B * M
    x2d = xp.reshape(rows, N)

    tm = min(tm, rows)
    grid_m = pl.cdiv(rows, tm)
    inner = grid_m // 2 if grid_m % 2 == 0 else grid_m
    outer = grid_m // inner

    out = pl.pallas_call(
        _gemm_kernel,
        out_shape=jax.ShapeDtypeStruct((rows, K), jnp.float32),
        grid=(outer, inner),
        in_specs=[
            pl.BlockSpec((tm, N), lambda i, j: (i * inner + j, 0)),
            pl.BlockSpec((K, N), lambda i, j: (0, 0)),
        ],
        out_specs=pl.BlockSpec((tm, K), lambda i, j: (i * inner + j, 0)),
        scratch_shapes=[pltpu.VMEM((K, N), jnp.bfloat16)],
        compiler_params=pltpu.CompilerParams(
            dimension_semantics=("parallel", "arbitrary"),
            vmem_limit_bytes=56 << 20,
        ),
    )(x2d, m_param)
    return out.reshape(B, M, K)


def kernel(xp, m_param):
    return _solve(xp, m_param)


# speedup vs baseline: 1.0279x; 1.0279x over previous
import functools

import jax
import jax.numpy as jnp
from jax import lax
from jax.experimental import pallas as pl
from jax.experimental.pallas import tpu as pltpu

_NC = 4
_DN = (((1,), (1,)), ((), ()))


def _make_kernel(rows, tm, K, N):
    nt = rows // tm
    ck = K // _NC
    h = K // 2

    def _kernel(x_hbm, w_hbm, o_hbm, wb_ref, wf_ref, xbuf, obuf,
                wsem, xsem, osem):
        def x_copy(t):
            return pltpu.make_async_copy(
                x_hbm.at[pl.ds(t * tm, tm), :], xbuf.at[t % 2], xsem.at[t % 2]
            )

        def o_copy(t):
            return pltpu.make_async_copy(
                obuf.at[t % 2], o_hbm.at[pl.ds(t * tm, tm), :], osem.at[t % 2]
            )

        def w_copy(c):
            return pltpu.make_async_copy(w_hbm.at[c], wf_ref.at[c], wsem.at[c])

        for c in range(_NC):
            w_copy(c).start()
        x_copy(0).start()
        x_copy(1).start()

        x_copy(0).wait()
        x_bf = xbuf[0].astype(jnp.bfloat16)
        for c in range(_NC):
            w_copy(c).wait()
            wb_ref[pl.ds(c * ck, ck), :] = wf_ref[c].astype(jnp.bfloat16)
            obuf[0, :, pl.ds(c * ck, ck)] = lax.dot_general(
                x_bf, wb_ref[pl.ds(c * ck, ck), :],
                dimension_numbers=_DN, preferred_element_type=jnp.float32,
            )
        o_copy(0).start()

        for t in range(1, nt):
            if t + 1 < nt:
                x_copy(t + 1).start()
            x_copy(t).wait()
            if t >= 2:
                o_copy(t - 2).wait()
            x_bf = xbuf[t % 2].astype(jnp.bfloat16)
            obuf[t % 2, :, :h] = lax.dot_general(
                x_bf, wb_ref[:h, :],
                dimension_numbers=_DN, preferred_element_type=jnp.float32,
            )
            obuf[t % 2, :, h:] = lax.dot_general(
                x_bf, wb_ref[h:, :],
                dimension_numbers=_DN, preferred_element_type=jnp.float32,
            )
            o_copy(t).start()

        o_copy(nt - 2).wait()
        o_copy(nt - 1).wait()

    return _kernel


@functools.partial(jax.jit, static_argnames=("tm",))
def _solve(xp, m_param, tm=512):
    B, M, N = xp.shape
    K = m_param.shape[0]
    rows = B * M
    x2d = xp.reshape(rows, N)
    w3 = m_param.reshape(_NC, K // _NC, N)

    out = pl.pallas_call(
        _make_kernel(rows, tm, K, N),
        out_shape=jax.ShapeDtypeStruct((rows, K), jnp.float32),
        grid=(1,),
        in_specs=[
            pl.BlockSpec(memory_space=pl.ANY),
            pl.BlockSpec(memory_space=pl.ANY),
        ],
        out_specs=pl.BlockSpec(memory_space=pl.ANY),
        scratch_shapes=[
            pltpu.VMEM((K, N), jnp.bfloat16),
            pltpu.VMEM((_NC, K // _NC, N), jnp.float32),
            pltpu.VMEM((2, tm, N), jnp.float32),
            pltpu.VMEM((2, tm, K), jnp.float32),
            pltpu.SemaphoreType.DMA((_NC,)),
            pltpu.SemaphoreType.DMA((2,)),
            pltpu.SemaphoreType.DMA((2,)),
        ],
        compiler_params=pltpu.CompilerParams(
            dimension_semantics=("arbitrary",),
            vmem_limit_bytes=56 << 20,
        ),
    )(x2d, w3)
    return out.reshape(B, M, K)


def kernel(xp, m_param):
    return _solve(xp, m_param)
